# Initial kernel scaffold; baseline (speedup 1.0000x reference)
import jax
import jax.numpy as jnp
from jax.experimental import pallas as pl


def kernel(x_piece, edge_index_piece, batch, W1, b1, W2, b2, W3, b3):
    raise NotImplementedError("write your pallas kernel here")


# speedup vs baseline: 6.2570x; 6.2570x over previous
import functools
import jax
import jax.numpy as jnp
from jax import lax
from jax.experimental import pallas as pl
from jax.experimental.pallas import tpu as pltpu
from jax.experimental.pallas import tpu_sc as plsc

N = 10000
E = 320000
D = 128

NC = 2
NS = 16
NW = NC * NS

N_PAD = 10240
ROWS_PER_TILE = N_PAD // NS
CHUNK = 128
J = 80
E_PAD = NW * CHUNK * J

_mesh = plsc.VectorSubcoreMesh(core_axis_name="c", subcore_axis_name="s",
                               num_cores=NC, num_subcores=NS)


def _agg_body(hp, zinit, src_h, dst_h, out, srcc, dstc, rows, acc, sem):
    c = lax.axis_index("c")
    s = lax.axis_index("s")
    wid = c * NS + s
    r0 = s * ROWS_PER_TILE

    @pl.when(c == 0)
    def _():
        pltpu.sync_copy(hp.at[pl.ds(r0, ROWS_PER_TILE)],
                        acc.at[pl.ds(r0, ROWS_PER_TILE)])

    @pl.when(c != 0)
    def _():
        pltpu.sync_copy(zinit.at[pl.ds(r0, ROWS_PER_TILE)],
                        acc.at[pl.ds(r0, ROWS_PER_TILE)])

    plsc.subcore_barrier()

    base = wid * (J * CHUNK)

    def step(j, carry):
        off = pl.multiple_of(base + j * CHUNK, CHUNK)
        pltpu.sync_copy(src_h.at[pl.ds(off, CHUNK)], srcc)
        pltpu.sync_copy(dst_h.at[pl.ds(off, CHUNK)], dstc)
        pltpu.async_copy(hp.at[srcc], rows, sem).wait()
        pltpu.sync_copy(rows, acc.at[dstc], add=True)
        return carry

    lax.fori_loop(0, J, step, 0, unroll=False)

    plsc.subcore_barrier()
    pltpu.sync_copy(acc.at[pl.ds(r0, ROWS_PER_TILE)],
                    out.at[c, pl.ds(r0, ROWS_PER_TILE)])


@functools.partial(
    pl.kernel,
    out_type=jax.ShapeDtypeStruct((NC, N_PAD, D), jnp.float32),
    mesh=_mesh,
    scratch_types=[
        pltpu.VMEM((CHUNK,), jnp.int32),
        pltpu.VMEM((CHUNK,), jnp.int32),
        pltpu.VMEM((CHUNK, D), jnp.float32),
        pltpu.VMEM_SHARED((N_PAD, D), jnp.float32),
        pltpu.SemaphoreType.DMA,
    ],
    name="gcn_sc_aggregate",
)
def _agg(hp, zinit, src_h, dst_h, out, srcc, dstc, rows, acc, sem):
    _agg_body(hp, zinit, src_h, dst_h, out, srcc, dstc, rows, acc, sem)


def _deg_body(dst_h, out, dstc, hist):
    c = lax.axis_index("c")
    s = lax.axis_index("s")
    wid = c * NS + s

    def zstep(i, carry):
        hist[pl.ds(i * 16, 16)] = jnp.zeros((16,), jnp.float32)
        return carry

    lax.fori_loop(0, N_PAD // 16, zstep, 0, unroll=False)

    base = wid * (J * CHUNK)
    ones16 = jnp.ones((16,), jnp.float32)

    def step(j, carry):
        off = pl.multiple_of(base + j * CHUNK, CHUNK)
        pltpu.sync_copy(dst_h.at[pl.ds(off, CHUNK)], dstc)
        for k in range(CHUNK // 16):
            idx = dstc[pl.ds(k * 16, 16)]
            plsc.addupdate_scatter(hist, [idx], ones16)
        return carry

    lax.fori_loop(0, J, step, 0, unroll=False)
    pltpu.sync_copy(hist, out.at[c, s])


@functools.partial(
    pl.kernel,
    out_type=jax.ShapeDtypeStruct((NC, NS, N_PAD), jnp.float32),
    mesh=_mesh,
    scratch_types=[
        pltpu.VMEM((CHUNK,), jnp.int32),
        pltpu.VMEM((N_PAD,), jnp.float32),
    ],
    name="gcn_sc_degree",
    compiler_params=pltpu.CompilerParams(needs_layout_passes=False),
)
def _deg(dst_h, out, dstc, hist):
    _deg_body(dst_h, out, dstc, hist)


_R = 2048


def _gelu_exact(u):
    return 0.5 * u * (1.0 + lax.erf(u * 0.7071067811865476))


def _dis_of(dref):
    return lax.rsqrt(jnp.sum(dref[...], axis=1, keepdims=True) + 1.0)


def _tc_first(x, W, degs_t):
    def body(x_ref, w_ref, dref, o_ref):
        dis = _dis_of(dref)
        h = jnp.dot(x_ref[...], w_ref[...], preferred_element_type=jnp.float32)
        o_ref[...] = h * dis

    return pl.pallas_call(
        body,
        grid=(N_PAD // _R,),
        in_specs=[
            pl.BlockSpec((_R, D), lambda i: (i, 0)),
            pl.BlockSpec((D, D), lambda i: (0, 0)),
            pl.BlockSpec((_R, NW), lambda i: (i, 0)),
        ],
        out_specs=pl.BlockSpec((_R, D), lambda i: (i, 0)),
        out_shape=jax.ShapeDtypeStruct((N_PAD, D), jnp.float32),
    )(x, W, degs_t)


def _tc_mid(a0, a1, degs_t, b, W):
    def body(a0r, a1r, dref, br, wr, o_ref):
        dis = _dis_of(dref)
        conv = (a0r[...] + a1r[...]) * dis + br[...]
        g = _gelu_exact(conv)
        h = jnp.dot(g, wr[...], preferred_element_type=jnp.float32)
        o_ref[...] = h * dis

    return pl.pallas_call(
        body,
        grid=(N_PAD // _R,),
        in_specs=[
            pl.BlockSpec((_R, D), lambda i: (i, 0)),
            pl.BlockSpec((_R, D), lambda i: (i, 0)),
            pl.BlockSpec((_R, NW), lambda i: (i, 0)),
            pl.BlockSpec((1, D), lambda i: (0, 0)),
            pl.BlockSpec((D, D), lambda i: (0, 0)),
        ],
        out_specs=pl.BlockSpec((_R, D), lambda i: (i, 0)),
        out_shape=jax.ShapeDtypeStruct((N_PAD, D), jnp.float32),
    )(a0, a1, degs_t, b, W)


def _tc_last(a0, a1, degs_t, b):
    def body(a0r, a1r, dref, br, o_ref):
        dis = _dis_of(dref)
        o_ref[...] = (a0r[...] + a1r[...]) * dis + br[...]

    return pl.pallas_call(
        body,
        grid=(N_PAD // _R,),
        in_specs=[
            pl.BlockSpec((_R, D), lambda i: (i, 0)),
            pl.BlockSpec((_R, D), lambda i: (i, 0)),
            pl.BlockSpec((_R, NW), lambda i: (i, 0)),
            pl.BlockSpec((1, D), lambda i: (0, 0)),
        ],
        out_specs=pl.BlockSpec((_R, D), lambda i: (i, 0)),
        out_shape=jax.ShapeDtypeStruct((N_PAD, D), jnp.float32),
    )(a0, a1, degs_t, b)


def kernel(x_piece, edge_index_piece, batch, W1, b1, W2, b2, W3, b3):
    del batch
    src = edge_index_piece[0].astype(jnp.int32)
    dst = edge_index_piece[1].astype(jnp.int32)
    pad = jnp.full((E_PAD - E,), N_PAD - 1, dtype=jnp.int32)
    src_f = jnp.concatenate([src, pad])
    dst_f = jnp.concatenate([dst, pad])

    x_pad = jnp.pad(x_piece, ((0, N_PAD - N), (0, 0)))
    zinit = jnp.zeros((N_PAD, D), jnp.float32)

    degp = _deg(dst_f)
    degs_t = jnp.transpose(degp.reshape(NW, N_PAD))

    hp1 = _tc_first(x_pad, W1, degs_t)
    a1 = _agg(hp1, zinit, src_f, dst_f)
    hp2 = _tc_mid(a1[0], a1[1], degs_t, b1.reshape(1, D), W2)
    a2 = _agg(hp2, zinit, src_f, dst_f)
    hp3 = _tc_mid(a2[0], a2[1], degs_t, b2.reshape(1, D), W3)
    a3 = _agg(hp3, zinit, src_f, dst_f)
    out = _tc_last(a3[0], a3[1], degs_t, b3.reshape(1, D))
    return out[:N]


# speedup vs baseline: 7.5328x; 1.2039x over previous
import functools
import jax
import jax.numpy as jnp
from jax import lax
from jax.experimental import pallas as pl
from jax.experimental.pallas import tpu as pltpu
from jax.experimental.pallas import tpu_sc as plsc

N = 10000
E = 320000
D = 128

NC = 2
NS = 16
NW = NC * NS

N_PAD = 10240
ROWS_PER_TILE = N_PAD // NS
CHUNK = 128
J = 80
E_PAD = NW * CHUNK * J

_mesh = plsc.VectorSubcoreMesh(core_axis_name="c", subcore_axis_name="s",
                               num_cores=NC, num_subcores=NS)


def _agg_body(hp, zinit, src2, dst_h, out, srcb, dstc0, dstc1, rows, acc,
              semg, sems, semi):
    c = lax.axis_index("c")
    s = lax.axis_index("s")
    wid = c * NS + s
    r0 = s * ROWS_PER_TILE

    @pl.when(c == 0)
    def _():
        pltpu.sync_copy(hp.at[pl.ds(r0, ROWS_PER_TILE)],
                        acc.at[pl.ds(r0, ROWS_PER_TILE)])

    @pl.when(c != 0)
    def _():
        pltpu.sync_copy(zinit.at[pl.ds(r0, ROWS_PER_TILE)],
                        acc.at[pl.ds(r0, ROWS_PER_TILE)])

    pltpu.sync_copy(src2.at[pl.ds(wid * J, J)], srcb)
    plsc.subcore_barrier()

    base = wid * (J * CHUNK)
    dstc = (dstc0, dstc1)

    def dst_off(j):
        return pl.multiple_of(base + j * CHUNK, CHUNK)

    for b in range(2):
        pltpu.async_copy(dst_h.at[pl.ds(dst_off(b), CHUNK)], dstc[b], semi[b])
        pltpu.async_copy(hp.at[srcb.at[b]], rows.at[b], semg[b])

    def outer(t, carry):
        j0 = (t - 1) * 2
        for b in range(2):
            pltpu.make_async_copy(hp.at[srcb.at[j0 + b]], rows.at[b],
                                  semg[b]).wait()
            pltpu.make_async_copy(dst_h.at[pl.ds(dst_off(j0 + b), CHUNK)],
                                  dstc[b], semi[b]).wait()
            pltpu.async_copy(rows.at[b], acc.at[dstc[b]], sems[b], add=True)
        for b in range(2):
            pltpu.make_async_copy(rows.at[b], acc.at[dstc[b]], sems[b]).wait()
            jn = t * 2 + b
            pltpu.async_copy(dst_h.at[pl.ds(dst_off(jn), CHUNK)], dstc[b],
                             semi[b])
            pltpu.async_copy(hp.at[srcb.at[jn]], rows.at[b], semg[b])
        return carry

    lax.fori_loop(1, J // 2, outer, 0, unroll=False)

    j0 = J - 2
    for b in range(2):
        pltpu.make_async_copy(hp.at[srcb.at[j0 + b]], rows.at[b],
                              semg[b]).wait()
        pltpu.make_async_copy(dst_h.at[pl.ds(dst_off(j0 + b), CHUNK)],
                              dstc[b], semi[b]).wait()
        pltpu.sync_copy(rows.at[b], acc.at[dstc[b]], add=True)

    plsc.subcore_barrier()
    pltpu.sync_copy(acc.at[pl.ds(r0, ROWS_PER_TILE)],
                    out.at[c, pl.ds(r0, ROWS_PER_TILE)])


@functools.partial(
    pl.kernel,
    out_type=jax.ShapeDtypeStruct((NC, N_PAD, D), jnp.float32),
    mesh=_mesh,
    scratch_types=[
        pltpu.VMEM((J, CHUNK), jnp.int32),
        pltpu.VMEM((CHUNK,), jnp.int32),
        pltpu.VMEM((CHUNK,), jnp.int32),
        pltpu.VMEM((2, CHUNK, D), jnp.float32),
        pltpu.VMEM_SHARED((N_PAD, D), jnp.float32),
        [pltpu.SemaphoreType.DMA, pltpu.SemaphoreType.DMA],
        [pltpu.SemaphoreType.DMA, pltpu.SemaphoreType.DMA],
        [pltpu.SemaphoreType.DMA, pltpu.SemaphoreType.DMA],
    ],
    name="gcn_sc_aggregate",
)
def _agg(hp, zinit, src2, dst_h, out, srcb, dstc0, dstc1, rows, acc,
         semg, sems, semi):
    _agg_body(hp, zinit, src2, dst_h, out, srcb, dstc0, dstc1, rows, acc,
              semg, sems, semi)


def _deg_body(dst_h, out, dstc, hist):
    c = lax.axis_index("c")
    s = lax.axis_index("s")
    wid = c * NS + s

    def zstep(i, carry):
        hist[pl.ds(i * 16, 16)] = jnp.zeros((16,), jnp.float32)
        return carry

    lax.fori_loop(0, N_PAD // 16, zstep, 0, unroll=False)

    base = wid * (J * CHUNK)
    ones16 = jnp.ones((16,), jnp.float32)

    def step(j, carry):
        off = pl.multiple_of(base + j * CHUNK, CHUNK)
        pltpu.sync_copy(dst_h.at[pl.ds(off, CHUNK)], dstc)
        for k in range(CHUNK // 16):
            idx = dstc[pl.ds(k * 16, 16)]
            plsc.addupdate_scatter(hist, [idx], ones16)
        return carry

    lax.fori_loop(0, J, step, 0, unroll=False)
    pltpu.sync_copy(hist, out.at[c, s])


@functools.partial(
    pl.kernel,
    out_type=jax.ShapeDtypeStruct((NC, NS, N_PAD), jnp.float32),
    mesh=_mesh,
    scratch_types=[
        pltpu.VMEM((CHUNK,), jnp.int32),
        pltpu.VMEM((N_PAD,), jnp.float32),
    ],
    name="gcn_sc_degree",
    compiler_params=pltpu.CompilerParams(needs_layout_passes=False),
)
def _deg(dst_h, out, dstc, hist):
    _deg_body(dst_h, out, dstc, hist)


_R = 2048


def _gelu_exact(u):
    return 0.5 * u * (1.0 + lax.erf(u * 0.7071067811865476))


def _dis_of(dref):
    return lax.rsqrt(jnp.sum(dref[...], axis=1, keepdims=True) + 1.0)


def _tc_first(x, W, degs_t):
    def body(x_ref, w_ref, dref, o_ref):
        dis = _dis_of(dref)
        h = jnp.dot(x_ref[...], w_ref[...], preferred_element_type=jnp.float32)
        o_ref[...] = h * dis

    return pl.pallas_call(
        body,
        grid=(N_PAD // _R,),
        in_specs=[
            pl.BlockSpec((_R, D), lambda i: (i, 0)),
            pl.BlockSpec((D, D), lambda i: (0, 0)),
            pl.BlockSpec((_R, NW), lambda i: (i, 0)),
        ],
        out_specs=pl.BlockSpec((_R, D), lambda i: (i, 0)),
        out_shape=jax.ShapeDtypeStruct((N_PAD, D), jnp.float32),
    )(x, W, degs_t)


def _tc_mid(a0, a1, degs_t, b, W):
    def body(a0r, a1r, dref, br, wr, o_ref):
        dis = _dis_of(dref)
        conv = (a0r[...] + a1r[...]) * dis + br[...]
        g = _gelu_exact(conv)
        h = jnp.dot(g, wr[...], preferred_element_type=jnp.float32)
        o_ref[...] = h * dis

    return pl.pallas_call(
        body,
        grid=(N_PAD // _R,),
        in_specs=[
            pl.BlockSpec((_R, D), lambda i: (i, 0)),
            pl.BlockSpec((_R, D), lambda i: (i, 0)),
            pl.BlockSpec((_R, NW), lambda i: (i, 0)),
            pl.BlockSpec((1, D), lambda i: (0, 0)),
            pl.BlockSpec((D, D), lambda i: (0, 0)),
        ],
        out_specs=pl.BlockSpec((_R, D), lambda i: (i, 0)),
        out_shape=jax.ShapeDtypeStruct((N_PAD, D), jnp.float32),
    )(a0, a1, degs_t, b, W)


def _tc_last(a0, a1, degs_t, b):
    def body(a0r, a1r, dref, br, o_ref):
        dis = _dis_of(dref)
        o_ref[...] = (a0r[...] + a1r[...]) * dis + br[...]

    return pl.pallas_call(
        body,
        grid=(N_PAD // _R,),
        in_specs=[
            pl.BlockSpec((_R, D), lambda i: (i, 0)),
            pl.BlockSpec((_R, D), lambda i: (i, 0)),
            pl.BlockSpec((_R, NW), lambda i: (i, 0)),
            pl.BlockSpec((1, D), lambda i: (0, 0)),
        ],
        out_specs=pl.BlockSpec((_R, D), lambda i: (i, 0)),
        out_shape=jax.ShapeDtypeStruct((N_PAD, D), jnp.float32),
    )(a0, a1, degs_t, b)


def kernel(x_piece, edge_index_piece, batch, W1, b1, W2, b2, W3, b3):
    del batch
    src = edge_index_piece[0].astype(jnp.int32)
    dst = edge_index_piece[1].astype(jnp.int32)
    pad = jnp.full((E_PAD - E,), N_PAD - 1, dtype=jnp.int32)
    src2 = jnp.concatenate([src, pad]).reshape(E_PAD // CHUNK, CHUNK)
    dst_f = jnp.concatenate([dst, pad])

    x_pad = jnp.pad(x_piece, ((0, N_PAD - N), (0, 0)))
    zinit = jnp.zeros((N_PAD, D), jnp.float32)

    degp = _deg(dst_f)
    degs_t = jnp.transpose(degp.reshape(NW, N_PAD))

    hp1 = _tc_first(x_pad, W1, degs_t)
    a1 = _agg(hp1, zinit, src2, dst_f)
    hp2 = _tc_mid(a1[0], a1[1], degs_t, b1.reshape(1, D), W2)
    a2 = _agg(hp2, zinit, src2, dst_f)
    hp3 = _tc_mid(a2[0], a2[1], degs_t, b2.reshape(1, D), W3)
    a3 = _agg(hp3, zinit, src2, dst_f)
    out = _tc_last(a3[0], a3[1], degs_t, b3.reshape(1, D))
    return out[:N]


# speedup vs baseline: 21.5290x; 2.8580x over previous
import functools
import jax
import jax.numpy as jnp
from jax import lax
from jax.experimental import pallas as pl
from jax.experimental.pallas import tpu as pltpu
from jax.experimental.pallas import tpu_sc as plsc

N = 10000
E = 320000
D = 128

NC = 2
NS = 16
NW = NC * NS

N_PAD = 10240
ROWS_PER_TILE = N_PAD // NS
CHUNK = 128
J = 80
E_PAD = NW * CHUNK * J

_mesh = plsc.VectorSubcoreMesh(core_axis_name="c", subcore_axis_name="s",
                               num_cores=NC, num_subcores=NS)


def _agg_body(hp, zinit, src2, dst_h, out, srcb, dstc0, dstc1, rows, acc,
              semg, sems, semi):
    c = lax.axis_index("c")
    s = lax.axis_index("s")
    wid = c * NS + s
    r0 = s * ROWS_PER_TILE

    @pl.when(c == 0)
    def _():
        pltpu.sync_copy(hp.at[pl.ds(r0, ROWS_PER_TILE)],
                        acc.at[pl.ds(r0, ROWS_PER_TILE)])

    @pl.when(c != 0)
    def _():
        pltpu.sync_copy(zinit.at[pl.ds(r0, ROWS_PER_TILE)],
                        acc.at[pl.ds(r0, ROWS_PER_TILE)])

    pltpu.sync_copy(src2.at[pl.ds(wid * J, J)], srcb)
    plsc.subcore_barrier()

    base = wid * (J * CHUNK)
    dstc = (dstc0, dstc1)

    def dst_off(j):
        return pl.multiple_of(base + j * CHUNK, CHUNK)

    for b in range(2):
        pltpu.async_copy(dst_h.at[pl.ds(dst_off(b), CHUNK)], dstc[b], semi[b])
        pltpu.async_copy(hp.at[srcb.at[b]], rows.at[b], semg[b])

    def outer(t, carry):
        j0 = (t - 1) * 2
        for b in range(2):
            pltpu.make_async_copy(hp.at[srcb.at[j0 + b]], rows.at[b],
                                  semg[b]).wait()
            pltpu.make_async_copy(dst_h.at[pl.ds(dst_off(j0 + b), CHUNK)],
                                  dstc[b], semi[b]).wait()
            pltpu.async_copy(rows.at[b], acc.at[dstc[b]], sems[b], add=True)
        for b in range(2):
            pltpu.make_async_copy(rows.at[b], acc.at[dstc[b]], sems[b]).wait()
            jn = t * 2 + b
            pltpu.async_copy(dst_h.at[pl.ds(dst_off(jn), CHUNK)], dstc[b],
                             semi[b])
            pltpu.async_copy(hp.at[srcb.at[jn]], rows.at[b], semg[b])
        return carry

    lax.fori_loop(1, J // 2, outer, 0, unroll=False)

    j0 = J - 2
    for b in range(2):
        pltpu.make_async_copy(hp.at[srcb.at[j0 + b]], rows.at[b],
                              semg[b]).wait()
        pltpu.make_async_copy(dst_h.at[pl.ds(dst_off(j0 + b), CHUNK)],
                              dstc[b], semi[b]).wait()
        pltpu.sync_copy(rows.at[b], acc.at[dstc[b]], add=True)

    plsc.subcore_barrier()
    pltpu.sync_copy(acc.at[pl.ds(r0, ROWS_PER_TILE)],
                    out.at[c, pl.ds(r0, ROWS_PER_TILE)])


@functools.partial(
    pl.kernel,
    out_type=jax.ShapeDtypeStruct((NC, N_PAD, D), jnp.float32),
    mesh=_mesh,
    scratch_types=[
        pltpu.VMEM((J, CHUNK), jnp.int32),
        pltpu.VMEM((CHUNK,), jnp.int32),
        pltpu.VMEM((CHUNK,), jnp.int32),
        pltpu.VMEM((2, CHUNK, D), jnp.float32),
        pltpu.VMEM_SHARED((N_PAD, D), jnp.float32),
        [pltpu.SemaphoreType.DMA, pltpu.SemaphoreType.DMA],
        [pltpu.SemaphoreType.DMA, pltpu.SemaphoreType.DMA],
        [pltpu.SemaphoreType.DMA, pltpu.SemaphoreType.DMA],
    ],
    name="gcn_sc_aggregate",
)
def _agg(hp, zinit, src2, dst_h, out, srcb, dstc0, dstc1, rows, acc,
         semg, sems, semi):
    _agg_body(hp, zinit, src2, dst_h, out, srcb, dstc0, dstc1, rows, acc,
              semg, sems, semi)


def _deg_body(dst_h, out, dstc, hist):
    c = lax.axis_index("c")
    s = lax.axis_index("s")
    wid = c * NS + s

    def zstep(i, carry):
        hist[pl.ds(i * 16, 16)] = jnp.zeros((16,), jnp.float32)
        return carry

    lax.fori_loop(0, N_PAD // 16, zstep, 0, unroll=False)

    base = wid * (J * CHUNK)
    ones16 = jnp.ones((16,), jnp.float32)

    def step(j, carry):
        off = pl.multiple_of(base + j * CHUNK, CHUNK)
        pltpu.sync_copy(dst_h.at[pl.ds(off, CHUNK)], dstc)
        for k in range(CHUNK // 16):
            idx = dstc[pl.ds(k * 16, 16)]
            plsc.addupdate_scatter(hist, [idx], ones16)
        return carry

    lax.fori_loop(0, J, step, 0, unroll=False)
    pltpu.sync_copy(hist, out.at[c, s])


@functools.partial(
    pl.kernel,
    out_type=jax.ShapeDtypeStruct((NC, NS, N_PAD), jnp.float32),
    mesh=_mesh,
    scratch_types=[
        pltpu.VMEM((CHUNK,), jnp.int32),
        pltpu.VMEM((N_PAD,), jnp.float32),
    ],
    name="gcn_sc_degree",
    compiler_params=pltpu.CompilerParams(needs_layout_passes=False),
)
def _deg(dst_h, out, dstc, hist):
    _deg_body(dst_h, out, dstc, hist)


_R = 2048


def _gelu_exact(u):
    return 0.5 * u * (1.0 + lax.erf(u * 0.7071067811865476))


def _dis_of(dref):
    return lax.rsqrt(jnp.sum(dref[...], axis=1, keepdims=True) + 1.0)


def _tc_first(x, W, degs_t):
    def body(x_ref, w_ref, dref, o_ref):
        dis = _dis_of(dref)
        h = jnp.dot(x_ref[...], w_ref[...], preferred_element_type=jnp.float32)
        o_ref[...] = h * dis

    return pl.pallas_call(
        body,
        grid=(N_PAD // _R,),
        in_specs=[
            pl.BlockSpec((_R, D), lambda i: (i, 0)),
            pl.BlockSpec((D, D), lambda i: (0, 0)),
            pl.BlockSpec((_R, NW), lambda i: (i, 0)),
        ],
        out_specs=pl.BlockSpec((_R, D), lambda i: (i, 0)),
        out_shape=jax.ShapeDtypeStruct((N_PAD, D), jnp.float32),
    )(x, W, degs_t)


def _tc_mid(a0, a1, degs_t, b, W):
    def body(a0r, a1r, dref, br, wr, o_ref):
        dis = _dis_of(dref)
        conv = (a0r[...] + a1r[...]) * dis + br[...]
        g = _gelu_exact(conv)
        h = jnp.dot(g, wr[...], preferred_element_type=jnp.float32)
        o_ref[...] = h * dis

    return pl.pallas_call(
        body,
        grid=(N_PAD // _R,),
        in_specs=[
            pl.BlockSpec((_R, D), lambda i: (i, 0)),
            pl.BlockSpec((_R, D), lambda i: (i, 0)),
            pl.BlockSpec((_R, NW), lambda i: (i, 0)),
            pl.BlockSpec((1, D), lambda i: (0, 0)),
            pl.BlockSpec((D, D), lambda i: (0, 0)),
        ],
        out_specs=pl.BlockSpec((_R, D), lambda i: (i, 0)),
        out_shape=jax.ShapeDtypeStruct((N_PAD, D), jnp.float32),
    )(a0, a1, degs_t, b, W)


def _tc_last(a0, a1, degs_t, b):
    def body(a0r, a1r, dref, br, o_ref):
        dis = _dis_of(dref)
        o_ref[...] = (a0r[...] + a1r[...]) * dis + br[...]

    return pl.pallas_call(
        body,
        grid=(N_PAD // _R,),
        in_specs=[
            pl.BlockSpec((_R, D), lambda i: (i, 0)),
            pl.BlockSpec((_R, D), lambda i: (i, 0)),
            pl.BlockSpec((_R, NW), lambda i: (i, 0)),
            pl.BlockSpec((1, D), lambda i: (0, 0)),
        ],
        out_specs=pl.BlockSpec((_R, D), lambda i: (i, 0)),
        out_shape=jax.ShapeDtypeStruct((N_PAD, D), jnp.float32),
    )(a0, a1, degs_t, b)


def kernel(x_piece, edge_index_piece, batch, W1, b1, W2, b2, W3, b3):
    del batch
    src = edge_index_piece[0].astype(jnp.int32)
    dst = edge_index_piece[1].astype(jnp.int32)
    pad = N + (jnp.arange(E_PAD - E, dtype=jnp.int32) % (N_PAD - N))
    src2 = jnp.concatenate([src, pad]).reshape(E_PAD // CHUNK, CHUNK)
    dst_f = jnp.concatenate([dst, pad])

    x_pad = jnp.pad(x_piece, ((0, N_PAD - N), (0, 0)))
    zinit = jnp.zeros((N_PAD, D), jnp.float32)

    degp = _deg(dst_f)
    degs_t = jnp.transpose(degp.reshape(NW, N_PAD))

    hp1 = _tc_first(x_pad, W1, degs_t)
    a1 = _agg(hp1, zinit, src2, dst_f)
    hp2 = _tc_mid(a1[0], a1[1], degs_t, b1.reshape(1, D), W2)
    a2 = _agg(hp2, zinit, src2, dst_f)
    hp3 = _tc_mid(a2[0], a2[1], degs_t, b2.reshape(1, D), W3)
    a3 = _agg(hp3, zinit, src2, dst_f)
    out = _tc_last(a3[0], a3[1], degs_t, b3.reshape(1, D))
    return out[:N]


# speedup vs baseline: 23.2076x; 1.0780x over previous
import functools
import jax
import jax.numpy as jnp
from jax import lax
from jax.experimental import pallas as pl
from jax.experimental.pallas import tpu as pltpu
from jax.experimental.pallas import tpu_sc as plsc

N = 10000
E = 320000
D = 128

NC = 2
NS = 16
NW = NC * NS

N_PAD = 10240
ROWS_PER_TILE = N_PAD // NS
CHUNK = 128
J = 80
E_PAD = NW * CHUNK * J

_mesh = plsc.VectorSubcoreMesh(core_axis_name="c", subcore_axis_name="s",
                               num_cores=NC, num_subcores=NS)


def _agg_body(hp, zinit, src2, dst_h, out, srcb, dstc0, dstc1, rows, acc,
              semg, sems, semi):
    c = lax.axis_index("c")
    s = lax.axis_index("s")
    wid = c * NS + s
    r0 = s * ROWS_PER_TILE

    pltpu.sync_copy(src2.at[pl.ds(wid * J, J)], srcb)

    base = wid * (J * CHUNK)
    dstc = (dstc0, dstc1)

    def dst_off(j):
        return pl.multiple_of(base + j * CHUNK, CHUNK)

    for b in range(2):
        pltpu.async_copy(dst_h.at[pl.ds(dst_off(b), CHUNK)], dstc[b], semi[b])
        pltpu.async_copy(hp.at[srcb.at[b]], rows.at[b], semg[b])

    @pl.when(c == 0)
    def _():
        pltpu.sync_copy(hp.at[pl.ds(r0, ROWS_PER_TILE)],
                        acc.at[pl.ds(r0, ROWS_PER_TILE)])

    @pl.when(c != 0)
    def _():
        pltpu.sync_copy(zinit.at[pl.ds(r0, ROWS_PER_TILE)],
                        acc.at[pl.ds(r0, ROWS_PER_TILE)])

    plsc.subcore_barrier()

    def outer(t, carry):
        j0 = (t - 1) * 2
        for b in range(2):
            pltpu.make_async_copy(hp.at[srcb.at[j0 + b]], rows.at[b],
                                  semg[b]).wait()
            pltpu.make_async_copy(dst_h.at[pl.ds(dst_off(j0 + b), CHUNK)],
                                  dstc[b], semi[b]).wait()
            pltpu.async_copy(rows.at[b], acc.at[dstc[b]], sems[b], add=True)
        for b in range(2):
            pltpu.make_async_copy(rows.at[b], acc.at[dstc[b]], sems[b]).wait()
            jn = t * 2 + b
            pltpu.async_copy(dst_h.at[pl.ds(dst_off(jn), CHUNK)], dstc[b],
                             semi[b])
            pltpu.async_copy(hp.at[srcb.at[jn]], rows.at[b], semg[b])
        return carry

    lax.fori_loop(1, J // 2, outer, 0, unroll=False)

    j0 = J - 2
    for b in range(2):
        pltpu.make_async_copy(hp.at[srcb.at[j0 + b]], rows.at[b],
                              semg[b]).wait()
        pltpu.make_async_copy(dst_h.at[pl.ds(dst_off(j0 + b), CHUNK)],
                              dstc[b], semi[b]).wait()
        pltpu.sync_copy(rows.at[b], acc.at[dstc[b]], add=True)

    plsc.subcore_barrier()
    pltpu.sync_copy(acc.at[pl.ds(r0, ROWS_PER_TILE)],
                    out.at[c, pl.ds(r0, ROWS_PER_TILE)])


@functools.partial(
    pl.kernel,
    out_type=jax.ShapeDtypeStruct((NC, N_PAD, D), jnp.float32),
    mesh=_mesh,
    scratch_types=[
        pltpu.VMEM((J, CHUNK), jnp.int32),
        pltpu.VMEM((CHUNK,), jnp.int32),
        pltpu.VMEM((CHUNK,), jnp.int32),
        pltpu.VMEM((2, CHUNK, D), jnp.float32),
        pltpu.VMEM_SHARED((N_PAD, D), jnp.float32),
        [pltpu.SemaphoreType.DMA, pltpu.SemaphoreType.DMA],
        [pltpu.SemaphoreType.DMA, pltpu.SemaphoreType.DMA],
        [pltpu.SemaphoreType.DMA, pltpu.SemaphoreType.DMA],
    ],
    name="gcn_sc_aggregate",
)
def _agg(hp, zinit, src2, dst_h, out, srcb, dstc0, dstc1, rows, acc,
         semg, sems, semi):
    _agg_body(hp, zinit, src2, dst_h, out, srcb, dstc0, dstc1, rows, acc,
              semg, sems, semi)


def _deg_body(dst2, out, dstb, hist):
    c = lax.axis_index("c")
    s = lax.axis_index("s")
    wid = c * NS + s

    pltpu.sync_copy(dst2.at[pl.ds(wid * J, J)], dstb)

    def zstep(i, carry):
        hist[pl.ds(i * 16, 16)] = jnp.zeros((16,), jnp.float32)
        return carry

    lax.fori_loop(0, N_PAD // 16, zstep, 0, unroll=False)

    ones16 = jnp.ones((16,), jnp.float32)

    def step(j, carry):
        for k in range(CHUNK // 16):
            idx = dstb[j, pl.ds(k * 16, 16)]
            plsc.addupdate_scatter(hist, [idx], ones16)
        return carry

    lax.fori_loop(0, J, step, 0, unroll=False)
    pltpu.sync_copy(hist, out.at[c, s])


@functools.partial(
    pl.kernel,
    out_type=jax.ShapeDtypeStruct((NC, NS, N_PAD), jnp.float32),
    mesh=_mesh,
    scratch_types=[
        pltpu.VMEM((J, CHUNK), jnp.int32),
        pltpu.VMEM((N_PAD,), jnp.float32),
    ],
    name="gcn_sc_degree",
    compiler_params=pltpu.CompilerParams(needs_layout_passes=False),
)
def _deg(dst2, out, dstb, hist):
    _deg_body(dst2, out, dstb, hist)


_R = 2048


def _gelu_exact(u):
    return 0.5 * u * (1.0 + lax.erf(u * 0.7071067811865476))


def _dis_of(dref):
    return lax.rsqrt(jnp.sum(dref[...], axis=1, keepdims=True) + 1.0)


def _tc_first(x, W, degs_t):
    def body(x_ref, w_ref, dref, o_ref):
        dis = _dis_of(dref)
        h = jnp.dot(x_ref[...], w_ref[...], preferred_element_type=jnp.float32)
        o_ref[...] = h * dis

    return pl.pallas_call(
        body,
        grid=(N_PAD // _R,),
        in_specs=[
            pl.BlockSpec((_R, D), lambda i: (i, 0)),
            pl.BlockSpec((D, D), lambda i: (0, 0)),
            pl.BlockSpec((_R, NW), lambda i: (i, 0)),
        ],
        out_specs=pl.BlockSpec((_R, D), lambda i: (i, 0)),
        out_shape=jax.ShapeDtypeStruct((N_PAD, D), jnp.float32),
    )(x, W, degs_t)


def _tc_mid(a0, a1, degs_t, b, W):
    def body(a0r, a1r, dref, br, wr, o_ref):
        dis = _dis_of(dref)
        conv = (a0r[...] + a1r[...]) * dis + br[...]
        g = _gelu_exact(conv)
        h = jnp.dot(g, wr[...], preferred_element_type=jnp.float32)
        o_ref[...] = h * dis

    return pl.pallas_call(
        body,
        grid=(N_PAD // _R,),
        in_specs=[
            pl.BlockSpec((_R, D), lambda i: (i, 0)),
            pl.BlockSpec((_R, D), lambda i: (i, 0)),
            pl.BlockSpec((_R, NW), lambda i: (i, 0)),
            pl.BlockSpec((1, D), lambda i: (0, 0)),
            pl.BlockSpec((D, D), lambda i: (0, 0)),
        ],
        out_specs=pl.BlockSpec((_R, D), lambda i: (i, 0)),
        out_shape=jax.ShapeDtypeStruct((N_PAD, D), jnp.float32),
    )(a0, a1, degs_t, b, W)


def _tc_last(a0, a1, degs_t, b):
    def body(a0r, a1r, dref, br, o_ref):
        dis = _dis_of(dref)
        o_ref[...] = (a0r[...] + a1r[...]) * dis + br[...]

    return pl.pallas_call(
        body,
        grid=(N_PAD // _R,),
        in_specs=[
            pl.BlockSpec((_R, D), lambda i: (i, 0)),
            pl.BlockSpec((_R, D), lambda i: (i, 0)),
            pl.BlockSpec((_R, NW), lambda i: (i, 0)),
            pl.BlockSpec((1, D), lambda i: (0, 0)),
        ],
        out_specs=pl.BlockSpec((_R, D), lambda i: (i, 0)),
        out_shape=jax.ShapeDtypeStruct((N_PAD, D), jnp.float32),
    )(a0, a1, degs_t, b)


def kernel(x_piece, edge_index_piece, batch, W1, b1, W2, b2, W3, b3):
    del batch
    src = edge_index_piece[0].astype(jnp.int32)
    dst = edge_index_piece[1].astype(jnp.int32)
    pad = N + (jnp.arange(E_PAD - E, dtype=jnp.int32) % (N_PAD - N))
    src2 = jnp.concatenate([src, pad]).reshape(E_PAD // CHUNK, CHUNK)
    dst_f = jnp.concatenate([dst, pad])

    x_pad = jnp.pad(x_piece, ((0, N_PAD - N), (0, 0)))
    zinit = jnp.zeros((N_PAD, D), jnp.float32)

    dst2 = dst_f.reshape(E_PAD // CHUNK, CHUNK)
    degp = _deg(dst2)
    degs_t = jnp.transpose(degp.reshape(NW, N_PAD))

    hp1 = _tc_first(x_pad, W1, degs_t)
    a1 = _agg(hp1, zinit, src2, dst_f)
    hp2 = _tc_mid(a1[0], a1[1], degs_t, b1.reshape(1, D), W2)
    a2 = _agg(hp2, zinit, src2, dst_f)
    hp3 = _tc_mid(a2[0], a2[1], degs_t, b2.reshape(1, D), W3)
    a3 = _agg(hp3, zinit, src2, dst_f)
    out = _tc_last(a3[0], a3[1], degs_t, b3.reshape(1, D))
    return out[:N]
